# baseline (speedup 1.0000x reference)
import functools

import jax
import jax.numpy as jnp
from jax import lax
from jax.experimental import pallas as pl
from jax.experimental.pallas import tpu as pltpu
from jax.experimental.pallas import tpu_sc as plsc

N_NODES = 10000
N_EDGES = 320000
EMB = 64

NC, NS = 2, 16
NW = NC * NS
EPW = N_EDGES // NW
C = 80
NCHUNK = EPW // C
CH = 3200
EPS = N_EDGES // NC
NCH2 = EPS // CH
CPT = 4
ACC = CPT * N_NODES

BE = 3200
NEB = N_EDGES // BE
NB = 2000
NNB = N_NODES // NB

_f32 = jnp.float32


def _relu(x):
    return jnp.maximum(x, 0.0)


def _dot(a, b):
    return jnp.dot(a, b, preferred_element_type=_f32)


def _node0_body(stat, dyn, Ws1, bs1, Ws2, bs2, Wd1, bd1, Wd2, bd2,
                Wsi, Wsj, Wdi, Wdj, hs_o, hd_o, u_o, v_o):
    hs = _relu(_dot(_relu(_dot(stat[...], Ws1[...]) + bs1[...]), Ws2[...]) + bs2[...])
    hd = _relu(_dot(_relu(_dot(dyn[...], Wd1[...]) + bd1[...]), Wd2[...]) + bd2[...])
    hs_o[...] = hs
    hd_o[...] = hd
    u_o[...] = jnp.concatenate([_dot(hs, Wsi[...]) + _dot(hd, Wdi[...]), hd], axis=1)
    v_o[...] = jnp.concatenate([_dot(hs, Wsj[...]) + _dot(hd, Wdj[...]), hd], axis=1)


def _row_blk(ncols):
    return pl.BlockSpec((NB, ncols), lambda i: (i, 0))


def _full(x):
    return pl.BlockSpec(x.shape, lambda i: tuple(0 for _ in x.shape))


def _node0(stat, dyn, ws):
    out = [jax.ShapeDtypeStruct((N_NODES, EMB), _f32),
           jax.ShapeDtypeStruct((N_NODES, EMB), _f32),
           jax.ShapeDtypeStruct((N_NODES, 2 * EMB), _f32),
           jax.ShapeDtypeStruct((N_NODES, 2 * EMB), _f32)]
    return pl.pallas_call(_node0_body, out_shape=out)(stat, dyn, *ws)


def _edge_enc_body(ef, We1, be1, We2, be2, W1e0, b1e0, W1e1, b1e1, r0_o, r1_o):
    h = _relu(_dot(ef[...], We1[...]) + be1[...])
    e = _relu(_dot(h, We2[...]) + be2[...])
    r0_o[...] = _dot(e, W1e0[...]) + b1e0[...]
    r1_o[...] = _dot(e, W1e1[...]) + b1e1[...]


def _edge_enc(ef, We1, be1, We2, be2, W1e0, b1e0, W1e1, b1e1):
    full = lambda s: pl.BlockSpec(s, lambda i: (0, 0))
    blk = pl.BlockSpec((BE, EMB), lambda i: (i, 0))
    return pl.pallas_call(
        _edge_enc_body,
        grid=(NEB,),
        in_specs=[pl.BlockSpec((BE, 4), lambda i: (i, 0)),
                  full(We1.shape), full(be1.shape), full(We2.shape),
                  full(be2.shape), full(W1e0.shape), full(b1e0.shape),
                  full(W1e1.shape), full(b1e1.shape)],
        out_specs=[blk, blk],
        out_shape=[jax.ShapeDtypeStruct((N_EDGES, EMB), _f32)] * 2,
    )(ef, We1, be1, We2, be2, W1e0, b1e0, W1e1, b1e1)


def _edge_mlp_body(g, r, d, W2, b2, o):
    h = _relu(g[...] + r[...])
    psi = _relu(_dot(h, W2[...]) + b2[...])
    o[...] = psi * d[...]


def _edge_mlp(G, R, D, W2, b2):
    full = lambda s: pl.BlockSpec(s, lambda i: (0, 0))
    return pl.pallas_call(
        _edge_mlp_body,
        grid=(NEB,),
        in_specs=[pl.BlockSpec((BE, EMB), lambda i: (i, 0)),
                  pl.BlockSpec((BE, EMB), lambda i: (i, 0)),
                  pl.BlockSpec((BE, EMB), lambda i: (i, 0)),
                  full(W2.shape), full(b2.shape)],
        out_specs=pl.BlockSpec((BE, EMB), lambda i: (i, 0)),
        out_shape=jax.ShapeDtypeStruct((N_EDGES, EMB), _f32),
    )(G, R, D, W2, b2)


def _agg_dot(aggB, Wperm):
    return lax.dot_general(aggB, Wperm, (((0,), (0,)), ((), ())),
                           preferred_element_type=_f32)


def _update_prep_body(hd, agg, Wl, Wsi, Wsj, Wdi, Wdj, hs, hd_o, u_o, v_o):
    hd1 = hd[...] + _agg_dot(agg[...], Wl[...])
    hd_o[...] = hd1
    u_o[...] = jnp.concatenate([_dot(hs[...], Wsi[...]) + _dot(hd1, Wdi[...]), hd1], axis=1)
    v_o[...] = jnp.concatenate([_dot(hs[...], Wsj[...]) + _dot(hd1, Wdj[...]), hd1], axis=1)


def _update_prep(hd, agg, Wl, Wsi, Wsj, Wdi, Wdj, hs):
    out = [jax.ShapeDtypeStruct((N_NODES, EMB), _f32),
           jax.ShapeDtypeStruct((N_NODES, 2 * EMB), _f32),
           jax.ShapeDtypeStruct((N_NODES, 2 * EMB), _f32)]
    return pl.pallas_call(_update_prep_body, out_shape=out)(
        hd, agg, Wl, Wsi, Wsj, Wdi, Wdj, hs)


def _update_decode_body(hd, agg, Wl, uh, Wcomb, bp, Wo1, bo1, Wo2, bo2, o):
    hd2 = hd[...] + _agg_dot(agg[...], Wl[...])
    t = _relu(_dot(hd2, Wo1[...]) + bo1[...])
    phi = _relu(_dot(t, Wo2[...]) + bo2[...])
    proj = _dot(uh[...], Wcomb[...]) + bp[...]
    o[...] = _relu(proj + phi)


def _update_decode(hd, agg, Wl, uh, Wcomb, bp, Wo1, bo1, Wo2, bo2):
    return pl.pallas_call(
        _update_decode_body,
        out_shape=jax.ShapeDtypeStruct((N_NODES, 2), _f32),
    )(hd, agg, Wl, uh, Wcomb, bp, Wo1, bo1, Wo2, bo2)


_MESH = dict(core_axis_name="c", subcore_axis_name="s")


def _sc_gather_body(u_hbm, v_hbm, s3, r3, g_hbm, d_hbm,
                    idxs, idxr, bu0, bu1, bv0, bv1, gb0, gb1, db0, db1,
                    su0, su1, sv0, sv1):
    wid = lax.axis_index("s") * NC + lax.axis_index("c")
    base = wid * EPW
    pltpu.sync_copy(s3.at[wid], idxs)
    pltpu.sync_copy(r3.at[wid], idxr)
    bu, bv, gb, db = (bu0, bu1), (bv0, bv1), (gb0, gb1), (db0, db1)
    su, sv = (su0, su1), (sv0, sv1)

    def issue(k, b):
        pltpu.async_copy(u_hbm.at[idxs.at[k]], bu[b], su[b])
        pltpu.async_copy(v_hbm.at[idxr.at[k]], bv[b], sv[b])

    def wait(k, b):
        pltpu.make_async_copy(u_hbm.at[idxs.at[k]], bu[b], su[b]).wait()
        pltpu.make_async_copy(v_hbm.at[idxr.at[k]], bv[b], sv[b]).wait()

    def compute_store(k, b):
        ub, vb, gbb, dbb = bu[b], bv[b], gb[b], db[b]

        @plsc.parallel_loop(0, C, unroll=4)
        def row(rr):
            for c4 in range(4):
                sl = pl.ds(c4 * 16, 16)
                s2 = pl.ds(EMB + c4 * 16, 16)
                gbb[rr, sl] = ub[rr, sl] + vb[rr, sl]
                dbb[rr, sl] = vb[rr, s2] - ub[rr, s2]
        pltpu.sync_copy(gbb, g_hbm.at[pl.ds(base + k * C, C)])
        pltpu.sync_copy(dbb, d_hbm.at[pl.ds(base + k * C, C)])

    issue(0, 0)

    def pair(i, _):
        k0 = i * 2
        issue(k0 + 1, 1)
        wait(k0, 0)
        compute_store(k0, 0)
        issue(k0 + 2, 0)
        wait(k0 + 1, 1)
        compute_store(k0 + 1, 1)
        return 0

    lax.fori_loop(0, (NCHUNK - 1) // 2, pair, 0)
    wait(NCHUNK - 1, 0)
    compute_store(NCHUNK - 1, 0)


def _sc_gather(U, V, s3, r3):
    f = pl.kernel(
        _sc_gather_body,
        out_type=[jax.ShapeDtypeStruct((N_EDGES, EMB), _f32),
                  jax.ShapeDtypeStruct((N_EDGES, EMB), _f32)],
        mesh=plsc.VectorSubcoreMesh(**_MESH),
        scratch_types=[pltpu.VMEM((NCHUNK, C), jnp.int32),
                       pltpu.VMEM((NCHUNK, C), jnp.int32)]
                      + [pltpu.VMEM((C, 2 * EMB), _f32)] * 4
                      + [pltpu.VMEM((C, EMB), _f32)] * 4
                      + [pltpu.SemaphoreType.DMA] * 4,
    )
    return f(U, V, s3, r3)


def _sc_scatter_body(mT, r2, out_hbm, ib0, ib1, mb0, mb1, acc,
                     sm0, sm1, si0, si1):
    cid = lax.axis_index("c")
    sid = lax.axis_index("s")
    wid = sid * NC + cid
    ebase = cid * EPS

    @plsc.parallel_loop(0, ACC // 16, unroll=8)
    def z(i):
        acc[pl.ds(i * 16, 16)] = jnp.zeros((16,), _f32)

    mb, ib, sm, si = (mb0, mb1), (ib0, ib1), (sm0, sm1), (si0, si1)

    def issue(k, b):
        pltpu.async_copy(mT.at[sid, :, pl.ds(ebase + k * CH, CH)], mb[b], sm[b])
        pltpu.async_copy(r2.at[cid, k], ib[b], si[b])

    def wait(k, b):
        pltpu.make_async_copy(mT.at[sid, :, pl.ds(ebase + k * CH, CH)],
                              mb[b], sm[b]).wait()
        pltpu.make_async_copy(r2.at[cid, k], ib[b], si[b]).wait()

    def proc(k, b):
        mbb, ibb = mb[b], ib[b]

        @plsc.parallel_loop(0, CH // 16, unroll=4)
        def g(i):
            sl = pl.ds(i * 16, 16)
            idx = ibb[sl]
            for c in range(CPT):
                plsc.addupdate_scatter(acc, [idx + (c * N_NODES)], mbb[c, sl])

    issue(0, 0)

    def pair(i, _):
        k0 = i * 2
        issue(k0 + 1, 1)
        wait(k0, 0)
        proc(k0, 0)
        issue(k0 + 2, 0)
        wait(k0 + 1, 1)
        proc(k0 + 1, 1)
        return 0

    lax.fori_loop(0, NCH2 // 2 - 1, pair, 0)
    issue(NCH2 - 1, 1)
    wait(NCH2 - 2, 0)
    proc(NCH2 - 2, 0)
    wait(NCH2 - 1, 1)
    proc(NCH2 - 1, 1)
    pltpu.sync_copy(acc, out_hbm.at[pl.ds(wid * ACC, ACC)])


def _sc_scatter(mT, r2):
    f = pl.kernel(
        _sc_scatter_body,
        out_type=jax.ShapeDtypeStruct((NW * ACC,), _f32),
        mesh=plsc.VectorSubcoreMesh(**_MESH),
        compiler_params=pltpu.CompilerParams(needs_layout_passes=False),
        scratch_types=[pltpu.VMEM((CH,), jnp.int32),
                       pltpu.VMEM((CH,), jnp.int32),
                       pltpu.VMEM((CPT, CH), _f32),
                       pltpu.VMEM((CPT, CH), _f32),
                       pltpu.VMEM((ACC,), _f32)]
                      + [pltpu.SemaphoreType.DMA] * 4,
    )
    return f(mT, r2).reshape(NW * CPT, N_NODES)


def kernel(static_node_features, dynamic_node_features, edge_features,
           U_history, edge_index, params):
    p = params
    (Ws1, bs1), (Ws2, bs2) = p['phi_s']
    (Wd1, bd1), (Wd2, bd2) = p['phi_d']
    (We1, be1), (We2, be2) = p['phi_e']
    row = lambda b: b.reshape(1, -1)

    layers = []
    for lyr in p['layers']:
        (W1, b1), (W2, b2) = lyr['psi']
        layers.append(dict(
            Wsi=W1[0:EMB], Wsj=W1[EMB:2 * EMB], Wdi=W1[2 * EMB:3 * EMB],
            Wdj=W1[3 * EMB:4 * EMB], We=W1[4 * EMB:5 * EMB], b1=b1,
            W2=W2, b2=b2, W=lyr['W']))


    senders = edge_index[0].reshape(NW, NCHUNK, C)
    receivers = edge_index[1].reshape(NW, NCHUNK, C)
    r2 = edge_index[1].reshape(NC, NCH2, CH)
    perm = jnp.array([CPT * (w // NC) + c for w in range(NW) for c in range(CPT)],
                     jnp.int32)

    Wp, bp = p['proj']
    Wcomb = _dot(jnp.kron(p['w_temp'], jnp.eye(3, dtype=_f32)), Wp)
    uh2d = U_history.reshape(N_NODES, -1)
    (Wo1, bo1), (Wo2, bo2) = p['phi_out']

    l0, l1 = layers
    hs, hd, Utab, Vtab = _node0(
        static_node_features, dynamic_node_features,
        (Ws1, row(bs1), Ws2, row(bs2), Wd1, row(bd1), Wd2, row(bd2),
         l0['Wsi'], l0['Wsj'], l0['Wdi'], l0['Wdj']))
    R0, R1 = _edge_enc(edge_features, We1, row(be1), We2, row(be2),
                       l0['We'], row(l0['b1']), l1['We'], row(l1['b1']))

    G, D = _sc_gather(Utab, Vtab, senders, receivers)
    msg = _edge_mlp(G, R0, D, l0['W2'], row(l0['b2']))
    agg = _sc_scatter(msg.T.reshape(NS, CPT, N_EDGES), r2)
    hd, Utab, Vtab = _update_prep(hd, agg, l0['W'][perm], l1['Wsi'], l1['Wsj'],
                                  l1['Wdi'], l1['Wdj'], hs)

    G, D = _sc_gather(Utab, Vtab, senders, receivers)
    msg = _edge_mlp(G, R1, D, l1['W2'], row(l1['b2']))
    agg = _sc_scatter(msg.T.reshape(NS, CPT, N_EDGES), r2)

    return _update_decode(hd, agg, l1['W'][perm], uh2d, Wcomb, row(bp),
                          Wo1, row(bo1), Wo2, row(bo2))


# speedup vs baseline: 1.2689x; 1.2689x over previous
import functools

import jax
import jax.numpy as jnp
from jax import lax
from jax.experimental import pallas as pl
from jax.experimental.pallas import tpu as pltpu
from jax.experimental.pallas import tpu_sc as plsc

N_NODES = 10000
N_EDGES = 320000
EMB = 64

NC, NS = 2, 16
NW = NC * NS
NH = 2
EH = N_EDGES // NH
EPW = EH // NW
C = 40
NCHUNK = EPW // C
CH = 3200
EPS = EH // NC
NCH2 = EPS // CH
CPT = 4
ACC = CPT * N_NODES

BE = 3200
NEB = EH // BE
NB = 2000
NNB = N_NODES // NB

_f32 = jnp.float32


def _relu(x):
    return jnp.maximum(x, 0.0)


def _dot(a, b):
    return jnp.dot(a, b, preferred_element_type=_f32)


def _node0_body(stat, dyn, Ws1, bs1, Ws2, bs2, Wd1, bd1, Wd2, bd2,
                Wsi, Wsj, Wdi, Wdj, hs_o, hd_o, u_o, v_o):
    hs = _relu(_dot(_relu(_dot(stat[...], Ws1[...]) + bs1[...]), Ws2[...]) + bs2[...])
    hd = _relu(_dot(_relu(_dot(dyn[...], Wd1[...]) + bd1[...]), Wd2[...]) + bd2[...])
    hs_o[...] = hs
    hd_o[...] = hd
    u_o[...] = jnp.concatenate([_dot(hs, Wsi[...]) + _dot(hd, Wdi[...]), hd], axis=1)
    v_o[...] = jnp.concatenate([_dot(hs, Wsj[...]) + _dot(hd, Wdj[...]), hd], axis=1)


def _row_blk(ncols):
    return pl.BlockSpec((NB, ncols), lambda i: (i, 0))


def _full(x):
    return pl.BlockSpec(x.shape, lambda i: tuple(0 for _ in x.shape))


def _node0(stat, dyn, ws):
    out = [jax.ShapeDtypeStruct((N_NODES, EMB), _f32),
           jax.ShapeDtypeStruct((N_NODES, EMB), _f32),
           jax.ShapeDtypeStruct((N_NODES, 2 * EMB), _f32),
           jax.ShapeDtypeStruct((N_NODES, 2 * EMB), _f32)]
    return pl.pallas_call(_node0_body, out_shape=out)(stat, dyn, *ws)


def _edge_enc_body(ef, We1, be1, We2, be2, W1e0, b1e0, W1e1, b1e1, r0_o, r1_o):
    h = _relu(_dot(ef[...], We1[...]) + be1[...])
    e = _relu(_dot(h, We2[...]) + be2[...])
    r0_o[...] = _dot(e, W1e0[...]) + b1e0[...]
    r1_o[...] = _dot(e, W1e1[...]) + b1e1[...]


def _edge_enc(ef, We1, be1, We2, be2, W1e0, b1e0, W1e1, b1e1):
    full = lambda s: pl.BlockSpec(s, lambda i: (0, 0))
    blk = pl.BlockSpec((BE, EMB), lambda i: (i, 0))
    return pl.pallas_call(
        _edge_enc_body,
        grid=(NEB,),
        in_specs=[pl.BlockSpec((BE, 4), lambda i: (i, 0)),
                  full(We1.shape), full(be1.shape), full(We2.shape),
                  full(be2.shape), full(W1e0.shape), full(b1e0.shape),
                  full(W1e1.shape), full(b1e1.shape)],
        out_specs=[blk, blk],
        out_shape=[jax.ShapeDtypeStruct((EH, EMB), _f32)] * 2,
    )(ef, We1, be1, We2, be2, W1e0, b1e0, W1e1, b1e1)


def _edge_mlp_body(g, r, d, W2, b2, o):
    h = _relu(g[...] + r[...])
    psi = _relu(_dot(h, W2[...]) + b2[...])
    m = (psi * d[...]).T
    o[...] = m.reshape(NS, CPT, BE)


def _edge_mlp(G, R, D, W2, b2):
    full = lambda s: pl.BlockSpec(s, lambda i: (0, 0))
    return pl.pallas_call(
        _edge_mlp_body,
        grid=(NEB,),
        in_specs=[pl.BlockSpec((BE, EMB), lambda i: (i, 0)),
                  pl.BlockSpec((BE, EMB), lambda i: (i, 0)),
                  pl.BlockSpec((BE, EMB), lambda i: (i, 0)),
                  full(W2.shape), full(b2.shape)],
        out_specs=pl.BlockSpec((NS, CPT, BE), lambda i: (0, 0, i)),
        out_shape=jax.ShapeDtypeStruct((NS, CPT, EH), _f32),
    )(G, R, D, W2, b2)


def _agg_dot(aggB, Wperm):
    return lax.dot_general(aggB, Wperm, (((0,), (0,)), ((), ())),
                           preferred_element_type=_f32)


def _update_prep_body(hd, agga, aggb, Wl, Wsi, Wsj, Wdi, Wdj, hs, hd_o, u_o, v_o):
    hd1 = hd[...] + _agg_dot(agga[...] + aggb[...], Wl[...])
    hd_o[...] = hd1
    u_o[...] = jnp.concatenate([_dot(hs[...], Wsi[...]) + _dot(hd1, Wdi[...]), hd1], axis=1)
    v_o[...] = jnp.concatenate([_dot(hs[...], Wsj[...]) + _dot(hd1, Wdj[...]), hd1], axis=1)


def _update_prep(hd, agg, Wl, Wsi, Wsj, Wdi, Wdj, hs):
    out = [jax.ShapeDtypeStruct((N_NODES, EMB), _f32),
           jax.ShapeDtypeStruct((N_NODES, 2 * EMB), _f32),
           jax.ShapeDtypeStruct((N_NODES, 2 * EMB), _f32)]
    return pl.pallas_call(_update_prep_body, out_shape=out)(
        hd, agg[0], agg[1], Wl, Wsi, Wsj, Wdi, Wdj, hs)


def _update_decode_body(hd, agga, aggb, Wl, uh, Wcomb, bp, Wo1, bo1, Wo2, bo2, o):
    hd2 = hd[...] + _agg_dot(agga[...] + aggb[...], Wl[...])
    t = _relu(_dot(hd2, Wo1[...]) + bo1[...])
    phi = _relu(_dot(t, Wo2[...]) + bo2[...])
    proj = _dot(uh[...], Wcomb[...]) + bp[...]
    o[...] = _relu(proj + phi)


def _update_decode(hd, agg, Wl, uh, Wcomb, bp, Wo1, bo1, Wo2, bo2):
    return pl.pallas_call(
        _update_decode_body,
        out_shape=jax.ShapeDtypeStruct((N_NODES, 2), _f32),
    )(hd, agg[0], agg[1], Wl, uh, Wcomb, bp, Wo1, bo1, Wo2, bo2)


_MESH = dict(core_axis_name="c", subcore_axis_name="s")


def _sc_gather_body(hbase, u_hbm, v_hbm, s3, r3, g_hbm, d_hbm,
                    idxs, idxr, bu0, bu1, bv0, bv1, gb0, gb1, db0, db1,
                    su0, su1, sv0, sv1):
    wid = lax.axis_index("s") * NC + lax.axis_index("c")
    base = wid * EPW
    pltpu.sync_copy(s3.at[hbase + wid], idxs)
    pltpu.sync_copy(r3.at[hbase + wid], idxr)
    bu, bv, gb, db = (bu0, bu1), (bv0, bv1), (gb0, gb1), (db0, db1)
    su, sv = (su0, su1), (sv0, sv1)

    def issue(k, b):
        pltpu.async_copy(u_hbm.at[idxs.at[k]], bu[b], su[b])
        pltpu.async_copy(v_hbm.at[idxr.at[k]], bv[b], sv[b])

    def wait(k, b):
        pltpu.make_async_copy(u_hbm.at[idxs.at[k]], bu[b], su[b]).wait()
        pltpu.make_async_copy(v_hbm.at[idxr.at[k]], bv[b], sv[b]).wait()

    def compute_store(k, b):
        ub, vb, gbb, dbb = bu[b], bv[b], gb[b], db[b]

        @plsc.parallel_loop(0, C, unroll=4)
        def row(rr):
            for c4 in range(4):
                sl = pl.ds(c4 * 16, 16)
                s2 = pl.ds(EMB + c4 * 16, 16)
                gbb[rr, sl] = ub[rr, sl] + vb[rr, sl]
                dbb[rr, sl] = vb[rr, s2] - ub[rr, s2]
        pltpu.sync_copy(gbb, g_hbm.at[pl.ds(base + k * C, C)])
        pltpu.sync_copy(dbb, d_hbm.at[pl.ds(base + k * C, C)])

    issue(0, 0)

    def pair(i, _):
        k0 = i * 2
        issue(k0 + 1, 1)
        wait(k0, 0)
        compute_store(k0, 0)
        issue(k0 + 2, 0)
        wait(k0 + 1, 1)
        compute_store(k0 + 1, 1)
        return 0

    lax.fori_loop(0, (NCHUNK - 1) // 2, pair, 0)
    wait(NCHUNK - 1, 0)
    compute_store(NCHUNK - 1, 0)


def _sc_gather(U, V, s3, r3, h):
    f = pl.kernel(
        functools.partial(_sc_gather_body, h * NW),
        out_type=[jax.ShapeDtypeStruct((EH, EMB), _f32),
                  jax.ShapeDtypeStruct((EH, EMB), _f32)],
        mesh=plsc.VectorSubcoreMesh(**_MESH),
        scratch_types=[pltpu.VMEM((NCHUNK, C), jnp.int32),
                       pltpu.VMEM((NCHUNK, C), jnp.int32)]
                      + [pltpu.VMEM((C, 2 * EMB), _f32)] * 4
                      + [pltpu.VMEM((C, EMB), _f32)] * 4
                      + [pltpu.SemaphoreType.DMA] * 4,
    )
    return f(U, V, s3, r3)


def _sc_scatter_body(hoff, mT, r2, out_hbm, ib0, ib1, mb0, mb1, acc,
                     sm0, sm1, si0, si1):
    cid = lax.axis_index("c")
    sid = lax.axis_index("s")
    wid = sid * NC + cid
    ebase = cid * EPS
    ibase = hoff + ebase

    @plsc.parallel_loop(0, ACC // 16, unroll=8)
    def z(i):
        acc[pl.ds(i * 16, 16)] = jnp.zeros((16,), _f32)

    mb, ib, sm, si = (mb0, mb1), (ib0, ib1), (sm0, sm1), (si0, si1)

    def issue(k, b):
        pltpu.async_copy(mT.at[sid, :, pl.ds(ebase + k * CH, CH)], mb[b], sm[b])
        pltpu.async_copy(r2.at[pl.ds(ibase + k * CH, CH)], ib[b], si[b])

    def wait(k, b):
        pltpu.make_async_copy(mT.at[sid, :, pl.ds(ebase + k * CH, CH)],
                              mb[b], sm[b]).wait()
        pltpu.make_async_copy(r2.at[pl.ds(ibase + k * CH, CH)], ib[b], si[b]).wait()

    def proc(k, b):
        mbb, ibb = mb[b], ib[b]

        @plsc.parallel_loop(0, CH // 16, unroll=4)
        def g(i):
            sl = pl.ds(i * 16, 16)
            idx = ibb[sl]
            for c in range(CPT):
                plsc.addupdate_scatter(acc, [idx + (c * N_NODES)], mbb[c, sl])

    issue(0, 0)

    def pair(i, _):
        k0 = i * 2
        issue(k0 + 1, 1)
        wait(k0, 0)
        proc(k0, 0)
        issue(k0 + 2, 0)
        wait(k0 + 1, 1)
        proc(k0 + 1, 1)
        return 0

    lax.fori_loop(0, (NCH2 - 1) // 2, pair, 0)
    wait(NCH2 - 1, 0)
    proc(NCH2 - 1, 0)
    pltpu.sync_copy(acc, out_hbm.at[pl.ds(wid * ACC, ACC)])


def _sc_scatter(mT, r2, h):
    f = pl.kernel(
        functools.partial(_sc_scatter_body, h * EH),
        out_type=jax.ShapeDtypeStruct((NW * ACC,), _f32),
        mesh=plsc.VectorSubcoreMesh(**_MESH),
        compiler_params=pltpu.CompilerParams(needs_layout_passes=False),
        scratch_types=[pltpu.VMEM((CH,), jnp.int32),
                       pltpu.VMEM((CH,), jnp.int32),
                       pltpu.VMEM((CPT, CH), _f32),
                       pltpu.VMEM((CPT, CH), _f32),
                       pltpu.VMEM((ACC,), _f32)]
                      + [pltpu.SemaphoreType.DMA] * 4,
    )
    return f(mT, r2).reshape(NW * CPT, N_NODES)


def kernel(static_node_features, dynamic_node_features, edge_features,
           U_history, edge_index, params):
    p = params
    (Ws1, bs1), (Ws2, bs2) = p['phi_s']
    (Wd1, bd1), (Wd2, bd2) = p['phi_d']
    (We1, be1), (We2, be2) = p['phi_e']
    row = lambda b: b.reshape(1, -1)

    layers = []
    for lyr in p['layers']:
        (W1, b1), (W2, b2) = lyr['psi']
        layers.append(dict(
            Wsi=W1[0:EMB], Wsj=W1[EMB:2 * EMB], Wdi=W1[2 * EMB:3 * EMB],
            Wdj=W1[3 * EMB:4 * EMB], We=W1[4 * EMB:5 * EMB], b1=b1,
            W2=W2, b2=b2, W=lyr['W']))


    senders3 = edge_index[0].reshape(NH * NW, NCHUNK, C)
    receivers3 = edge_index[1].reshape(NH * NW, NCHUNK, C)
    rflat = edge_index[1]
    ef2 = [lax.slice(edge_features, (h * EH, 0), ((h + 1) * EH, 4))
           for h in range(NH)]
    perm = jnp.array([CPT * (w // NC) + c for w in range(NW) for c in range(CPT)],
                     jnp.int32)

    Wp, bp = p['proj']
    Wcomb = _dot(jnp.kron(p['w_temp'], jnp.eye(3, dtype=_f32)), Wp)
    uh2d = U_history.reshape(N_NODES, -1)
    (Wo1, bo1), (Wo2, bo2) = p['phi_out']

    l0, l1 = layers
    hs, hd, Utab, Vtab = _node0(
        static_node_features, dynamic_node_features,
        (Ws1, row(bs1), Ws2, row(bs2), Wd1, row(bd1), Wd2, row(bd2),
         l0['Wsi'], l0['Wsj'], l0['Wdi'], l0['Wdj']))
    R0 = [None] * NH
    R1 = [None] * NH
    for h in range(NH):
        R0[h], R1[h] = _edge_enc(ef2[h], We1, row(be1), We2, row(be2),
                                 l0['We'], row(l0['b1']), l1['We'], row(l1['b1']))

    def _layer(Ut, Vt, Rl, W2, b2):
        agg = [None] * NH
        for h in range(NH):
            G, D = _sc_gather(Ut, Vt, senders3, receivers3, h)
            mT = _edge_mlp(G, Rl[h], D, W2, b2)
            agg[h] = _sc_scatter(mT, rflat, h)
        return agg

    agg = _layer(Utab, Vtab, R0, l0['W2'], row(l0['b2']))
    hd, Utab, Vtab = _update_prep(hd, agg, l0['W'][perm], l1['Wsi'], l1['Wsj'],
                                  l1['Wdi'], l1['Wdj'], hs)

    agg = _layer(Utab, Vtab, R1, l1['W2'], row(l1['b2']))

    return _update_decode(hd, agg, l1['W'][perm], uh2d, Wcomb, row(bp),
                          Wo1, row(bo1), Wo2, row(bo2))
